# Initial kernel scaffold; baseline (speedup 1.0000x reference)
import jax
import jax.numpy as jnp
from jax.experimental import pallas as pl


def kernel(data, year_table, month_table, day_table, hour_table):
    raise NotImplementedError("write your pallas kernel here")


# speedup vs baseline: 7.7342x; 7.7342x over previous
import functools

import jax
import jax.numpy as jnp
from jax.experimental import pallas as pl
from jax.experimental.pallas import tpu as pltpu

_ROWS = (2, 13, 31, 24)
_DIMS = (1, 6, 12, 10)
_ROW_OFF = (0, 2, 15, 46)
_COL_OFF = (0, 1, 7, 19)
_TOT_ROWS = 70
_EMB = 29
_F = 20


def _emb_kernel(x_ref, t_ref, o_ref):
    x = x_ref[...]
    bm = x.shape[0]
    col = jax.lax.broadcasted_iota(jnp.int32, (bm, _TOT_ROWS), 1)
    idx = x[:, 16:20].astype(jnp.int32)
    s = None
    for f in range(4):
        hit = (col == (idx[:, f:f + 1] + _ROW_OFF[f])).astype(jnp.float32)
        s = hit if s is None else s + hit
    emb = jax.lax.dot_general(
        s, t_ref[...], (((1,), (0,)), ((), ())),
        preferred_element_type=jnp.float32)
    o_ref[...] = jnp.concatenate([x, emb], axis=1)


def kernel(data, year_table, month_table, day_table, hour_table):
    b, l, f = data.shape
    n = b * l
    x = data.reshape(n, f)

    t = jnp.zeros((_TOT_ROWS, _EMB), jnp.float32)
    for tab, ro, co, d in zip(
        (year_table, month_table, day_table, hour_table),
        _ROW_OFF, _COL_OFF, _DIMS):
        t = jax.lax.dynamic_update_slice(t, tab, (ro, co))

    bm = 2048
    out = pl.pallas_call(
        _emb_kernel,
        grid=(n // bm,),
        in_specs=[
            pl.BlockSpec((bm, f), lambda i: (i, 0)),
            pl.BlockSpec((_TOT_ROWS, _EMB), lambda i: (0, 0)),
        ],
        out_specs=pl.BlockSpec((bm, f + _EMB), lambda i: (i, 0)),
        out_shape=jax.ShapeDtypeStruct((n, f + _EMB), jnp.float32),
    )(x, t)
    return out.reshape(b, l, f + _EMB)
